# Initial kernel scaffold; baseline (speedup 1.0000x reference)
import jax
import jax.numpy as jnp
from jax.experimental import pallas as pl


def kernel(ba1, bf1, raw_bf1, ba2, bf2, raw_bf2, W_enc, b_enc, a_enc, W_dec, b_dec, a_dec, Wb1, bb1, Wb2, bb2):
    raise NotImplementedError("write your pallas kernel here")


# speedup vs baseline: 1.8464x; 1.8464x over previous
import jax
import jax.numpy as jnp
from jax.experimental import pallas as pl
from jax.experimental.pallas import tpu as pltpu

B = 10000
S1 = 5
N_IN = 256
N_H = 64
BB = 400
NB = B // BB


def _prelu(x, a):
    return jnp.where(x >= 0, x, a * x)


def _view(i, ba_ref, bf_ref, raw_ref, ba_t_ref, bf_t_ref,
          W_enc, b_enc, a_enc, W_dec, b_dec, a_dec, Wb, bb, sc_ref):
    fts = [jnp.dot(bf_ref[:, m, :], W_enc, preferred_element_type=jnp.float32)
           for m in range(S1)]
    h = []
    for n in range(S1):
        acc = ba_ref[:, n * S1:n * S1 + 1] * fts[0]
        for m in range(1, S1):
            acc = acc + ba_ref[:, n * S1 + m:n * S1 + m + 1] * fts[m]
        h.append(_prelu(acc + b_enc, a_enc))
    c = (h[0] + h[1] + h[2] + h[3]) * 0.25
    hmv = h[S1 - 1]
    u = jnp.dot(hmv, Wb, preferred_element_type=jnp.float32)
    pos = jnp.sum(u * c, axis=1, keepdims=True) + bb
    fts_t = jnp.dot(bf_t_ref[...], W_enc, preferred_element_type=jnp.float32)
    h_t = _prelu(jnp.dot(ba_t_ref[...], fts_t,
                         preferred_element_type=jnp.float32) + b_enc, a_enc)
    c_tail = jnp.mean(h_t[0:S1 - 1, :], axis=0, keepdims=True)
    prev = jnp.where(i == 0, c_tail, sc_ref[...])
    c_shift = jnp.concatenate([prev, c[:BB - 1, :]], axis=0)
    neg = jnp.sum(u * c_shift, axis=1, keepdims=True) + bb
    sc_ref[...] = c[BB - 1:BB, :]
    g = ba_ref[:, (S1 - 1) * S1:(S1 - 1) * S1 + 1] * h[0]
    for n in range(1, S1):
        g = g + ba_ref[:, (S1 - 1) * S1 + n:(S1 - 1) * S1 + n + 1] * h[n]
    d = _prelu(jnp.dot(g, W_dec, preferred_element_type=jnp.float32) + b_dec,
               a_dec)
    diff = d - raw_ref[...]
    score = jnp.sqrt(jnp.sum(diff * diff, axis=1, keepdims=True) + 1e-12)
    return pos, neg, score


def _body(ba1_ref, bf1_ref, raw2_ref, ba2_ref, bf2_ref, raw1_ref,
          ba1t_ref, bf1t_ref, ba2t_ref, bf2t_ref,
          We_ref, be_ref, ae_ref, Wd_ref, bd_ref, ad_ref,
          Wb1_ref, bb1_ref, Wb2_ref, bb2_ref,
          pos1_ref, neg1_ref, pos2_ref, neg2_ref, s1_ref, s2_ref,
          sc1_ref, sc2_ref):
    i = pl.program_id(0)
    We = We_ref[...]
    be = be_ref[...]
    ae = ae_ref[0, 0]
    Wd = Wd_ref[...]
    bd = bd_ref[...]
    ad = ad_ref[0, 0]
    pos1, neg1, score1 = _view(i, ba1_ref, bf1_ref, raw2_ref, ba1t_ref,
                               bf1t_ref, We, be, ae, Wd, bd, ad,
                               Wb1_ref[...], bb1_ref[0, 0], sc1_ref)
    pos2, neg2, score2 = _view(i, ba2_ref, bf2_ref, raw1_ref, ba2t_ref,
                               bf2t_ref, We, be, ae, Wd, bd, ad,
                               Wb2_ref[...], bb2_ref[0, 0], sc2_ref)
    pos1_ref[...] = pos1
    neg1_ref[...] = neg1
    pos2_ref[...] = pos2
    neg2_ref[...] = neg2
    s1_ref[...] = score1
    s2_ref[...] = score2


def kernel(ba1, bf1, raw_bf1, ba2, bf2, raw_bf2,
           W_enc, b_enc, a_enc, W_dec, b_dec, a_dec,
           Wb1, bb1, Wb2, bb2):
    f32 = jnp.float32
    ba1_2d = ba1.reshape(B, S1 * S1)
    ba2_2d = ba2.reshape(B, S1 * S1)
    raw1_last = raw_bf1[:, -1, :]
    raw2_last = raw_bf2[:, -1, :]
    ba1_t = ba1[-1]
    bf1_t = bf1[-1]
    ba2_t = ba2[-1]
    bf2_t = bf2[-1]
    be = b_enc.reshape(1, N_H)
    bd = b_dec.reshape(1, N_IN)
    ae = jnp.asarray(a_enc, f32).reshape(1, 1)
    ad = jnp.asarray(a_dec, f32).reshape(1, 1)
    b1 = jnp.asarray(bb1, f32).reshape(1, 1)
    b2 = jnp.asarray(bb2, f32).reshape(1, 1)

    blk = lambda i: (i, 0)
    blk3 = lambda i: (i, 0, 0)
    rep = lambda i: (0, 0)
    rep3 = lambda i: (0, 0, 0)
    in_specs = [
        pl.BlockSpec((BB, S1 * S1), blk),
        pl.BlockSpec((BB, S1, N_IN), blk3),
        pl.BlockSpec((BB, N_IN), blk),
        pl.BlockSpec((BB, S1 * S1), blk),
        pl.BlockSpec((BB, S1, N_IN), blk3),
        pl.BlockSpec((BB, N_IN), blk),
        pl.BlockSpec((S1, S1), rep),
        pl.BlockSpec((S1, N_IN), rep),
        pl.BlockSpec((S1, S1), rep),
        pl.BlockSpec((S1, N_IN), rep),
        pl.BlockSpec((N_IN, N_H), rep),
        pl.BlockSpec((1, N_H), rep),
        pl.BlockSpec((1, 1), rep),
        pl.BlockSpec((N_H, N_IN), rep),
        pl.BlockSpec((1, N_IN), rep),
        pl.BlockSpec((1, 1), rep),
        pl.BlockSpec((N_H, N_H), rep),
        pl.BlockSpec((1, 1), rep),
        pl.BlockSpec((N_H, N_H), rep),
        pl.BlockSpec((1, 1), rep),
    ]
    out_specs = [pl.BlockSpec((BB, 1), blk)] * 6
    out_shape = [jax.ShapeDtypeStruct((B, 1), f32)] * 6

    pos1, neg1, pos2, neg2, score1, score2 = pl.pallas_call(
        _body,
        grid=(NB,),
        in_specs=in_specs,
        out_specs=out_specs,
        out_shape=out_shape,
        scratch_shapes=[pltpu.VMEM((1, N_H), f32), pltpu.VMEM((1, N_H), f32)],
    )(ba1_2d, bf1, raw2_last, ba2_2d, bf2, raw1_last,
      ba1_t, bf1_t, ba2_t, bf2_t,
      W_enc, be, ae, W_dec, bd, ad, Wb1, b1, Wb2, b2)

    return jnp.concatenate([
        pos1[:, 0], neg1[:, 0], pos2[:, 0], neg2[:, 0],
        score1[:, 0], score2[:, 0]], axis=0)


# speedup vs baseline: 2.1536x; 1.1664x over previous
import jax
import jax.numpy as jnp
from jax.experimental import pallas as pl
from jax.experimental.pallas import tpu as pltpu

B = 10000
S1 = 5
N_IN = 256
N_H = 64
BB = 400
NB = B // BB


def _prelu(x, a):
    return jnp.where(x >= 0, x, a * x)


def _view(i, ba_ref, bf_ref, raw_ref, ba_t_ref, bf_t_ref, E,
          W_enc, b_enc, a_enc, W_dec, b_dec, a_dec, Wb, bb, sc_ref):
    fts = [jnp.dot(bf_ref[:, m * N_IN:(m + 1) * N_IN], W_enc,
                   preferred_element_type=jnp.float32)
           for m in range(S1)]
    C = jnp.dot(ba_ref[...], E, preferred_element_type=jnp.float32)
    coef = lambda k: C[:, 128 * k:128 * k + N_H]
    h = []
    for n in range(S1):
        acc = coef(n * S1) * fts[0]
        for m in range(1, S1):
            acc = acc + coef(n * S1 + m) * fts[m]
        h.append(_prelu(acc + b_enc, a_enc))
    c = (h[0] + h[1] + h[2] + h[3]) * 0.25
    hmv = h[S1 - 1]
    u = jnp.dot(hmv, Wb, preferred_element_type=jnp.float32)
    pos = jnp.sum(u * c, axis=1, keepdims=True) + bb
    fts_t = jnp.dot(bf_t_ref[...], W_enc, preferred_element_type=jnp.float32)
    h_t = _prelu(jnp.dot(ba_t_ref[...], fts_t,
                         preferred_element_type=jnp.float32) + b_enc, a_enc)
    c_tail = jnp.mean(h_t[0:S1 - 1, :], axis=0, keepdims=True)
    prev = jnp.where(i == 0, c_tail, sc_ref[...])
    c_shift = jnp.concatenate([prev, c[:BB - 1, :]], axis=0)
    neg = jnp.sum(u * c_shift, axis=1, keepdims=True) + bb
    sc_ref[...] = c[BB - 1:BB, :]
    g = coef((S1 - 1) * S1) * h[0]
    for n in range(1, S1):
        g = g + coef((S1 - 1) * S1 + n) * h[n]
    d = _prelu(jnp.dot(g, W_dec, preferred_element_type=jnp.float32) + b_dec,
               a_dec)
    diff = d - raw_ref[...]
    score = jnp.sqrt(jnp.sum(diff * diff, axis=1, keepdims=True) + 1e-12)
    return pos, neg, score


def _body(ba1_ref, bf1_ref, raw2_ref, ba2_ref, bf2_ref, raw1_ref,
          ba1t_ref, bf1t_ref, ba2t_ref, bf2t_ref, E_ref,
          We_ref, be_ref, ae_ref, Wd_ref, bd_ref, ad_ref,
          Wb1_ref, bb1_ref, Wb2_ref, bb2_ref,
          pos1_ref, neg1_ref, pos2_ref, neg2_ref, s1_ref, s2_ref,
          sc1_ref, sc2_ref):
    i = pl.program_id(0)
    E = E_ref[...]
    We = We_ref[...]
    be = be_ref[...]
    ae = ae_ref[0, 0]
    Wd = Wd_ref[...]
    bd = bd_ref[...]
    ad = ad_ref[0, 0]
    pos1, neg1, score1 = _view(i, ba1_ref, bf1_ref, raw2_ref, ba1t_ref,
                               bf1t_ref, E, We, be, ae, Wd, bd, ad,
                               Wb1_ref[...], bb1_ref[0, 0], sc1_ref)
    pos2, neg2, score2 = _view(i, ba2_ref, bf2_ref, raw1_ref, ba2t_ref,
                               bf2t_ref, E, We, be, ae, Wd, bd, ad,
                               Wb2_ref[...], bb2_ref[0, 0], sc2_ref)
    pos1_ref[...] = pos1
    neg1_ref[...] = neg1
    pos2_ref[...] = pos2
    neg2_ref[...] = neg2
    s1_ref[...] = score1
    s2_ref[...] = score2


def kernel(ba1, bf1, raw_bf1, ba2, bf2, raw_bf2,
           W_enc, b_enc, a_enc, W_dec, b_dec, a_dec,
           Wb1, bb1, Wb2, bb2):
    f32 = jnp.float32
    ba1_2d = ba1.reshape(B, S1 * S1)
    ba2_2d = ba2.reshape(B, S1 * S1)
    bf1_w = bf1.reshape(B, S1 * N_IN)
    bf2_w = bf2.reshape(B, S1 * N_IN)
    raw1_last = raw_bf1[:, -1, :]
    raw2_last = raw_bf2[:, -1, :]
    ba1_t = ba1[-1]
    bf1_t = bf1[-1]
    ba2_t = ba2[-1]
    bf2_t = bf2[-1]
    E = (jnp.arange(S1 * S1, dtype=jnp.int32)[:, None] * 128
         <= jnp.arange(S1 * S1 * 128, dtype=jnp.int32)[None, :])
    E = (E & (jnp.arange(S1 * S1 * 128, dtype=jnp.int32)[None, :]
              < jnp.arange(S1 * S1, dtype=jnp.int32)[:, None] * 128 + N_H)
         ).astype(f32)
    be = b_enc.reshape(1, N_H)
    bd = b_dec.reshape(1, N_IN)
    ae = jnp.asarray(a_enc, f32).reshape(1, 1)
    ad = jnp.asarray(a_dec, f32).reshape(1, 1)
    b1 = jnp.asarray(bb1, f32).reshape(1, 1)
    b2 = jnp.asarray(bb2, f32).reshape(1, 1)

    blk = lambda i: (i, 0)
    blk3 = lambda i: (i, 0, 0)
    rep = lambda i: (0, 0)
    rep3 = lambda i: (0, 0, 0)
    in_specs = [
        pl.BlockSpec((BB, S1 * S1), blk),
        pl.BlockSpec((BB, S1 * N_IN), blk),
        pl.BlockSpec((BB, N_IN), blk),
        pl.BlockSpec((BB, S1 * S1), blk),
        pl.BlockSpec((BB, S1 * N_IN), blk),
        pl.BlockSpec((BB, N_IN), blk),
        pl.BlockSpec((S1, S1), rep),
        pl.BlockSpec((S1, N_IN), rep),
        pl.BlockSpec((S1, S1), rep),
        pl.BlockSpec((S1, N_IN), rep),
        pl.BlockSpec((S1 * S1, S1 * S1 * 128), rep),
        pl.BlockSpec((N_IN, N_H), rep),
        pl.BlockSpec((1, N_H), rep),
        pl.BlockSpec((1, 1), rep),
        pl.BlockSpec((N_H, N_IN), rep),
        pl.BlockSpec((1, N_IN), rep),
        pl.BlockSpec((1, 1), rep),
        pl.BlockSpec((N_H, N_H), rep),
        pl.BlockSpec((1, 1), rep),
        pl.BlockSpec((N_H, N_H), rep),
        pl.BlockSpec((1, 1), rep),
    ]
    out_specs = [pl.BlockSpec((BB, 1), blk)] * 6
    out_shape = [jax.ShapeDtypeStruct((B, 1), f32)] * 6

    pos1, neg1, pos2, neg2, score1, score2 = pl.pallas_call(
        _body,
        grid=(NB,),
        in_specs=in_specs,
        out_specs=out_specs,
        out_shape=out_shape,
        scratch_shapes=[pltpu.VMEM((1, N_H), f32), pltpu.VMEM((1, N_H), f32)],
    )(ba1_2d, bf1_w, raw2_last, ba2_2d, bf2_w, raw1_last,
      ba1_t, bf1_t, ba2_t, bf2_t, E,
      W_enc, be, ae, W_dec, bd, ad, Wb1, b1, Wb2, b2)

    return jnp.concatenate([
        pos1[:, 0], neg1[:, 0], pos2[:, 0], neg2[:, 0],
        score1[:, 0], score2[:, 0]], axis=0)
